# baseline (speedup 1.0000x reference)
import functools
import jax
import jax.numpy as jnp
from jax import lax
from jax.experimental import pallas as pl
from jax.experimental.pallas import tpu as pltpu
from jax.experimental.pallas import tpu_sc as plsc

B = 4
S = 8192
HID = 1024
NC = 2
NS = 16
NW = NC * NS
SW = S // NW
K = 8
NCH = SW // K
NBUF = 2
LANES = 16
SCALE = 32.0

_mesh = plsc.VectorSubcoreMesh(core_axis_name="c", subcore_axis_name="s")


@functools.partial(
    pl.kernel,
    out_type=jax.ShapeDtypeStruct((B, S, HID), jnp.float32),
    mesh=_mesh,
    scratch_types=[
        pltpu.VMEM((NCH * B * K,), jnp.int32),
        pltpu.VMEM((NBUF, K, HID), jnp.float32),
        pltpu.VMEM((NBUF, B * K, HID), jnp.float32),
        pltpu.SemaphoreType.DMA,
        pltpu.SemaphoreType.DMA,
        pltpu.SemaphoreType.DMA,
    ],
)
def _embed(ids_hbm, table_hbm, pe_hbm, out_hbm, idx_v, pe_v, rows_v,
           gsem, psem, osem):
    wid = lax.axis_index("s") * NC + lax.axis_index("c")
    base = wid * SW

    pltpu.sync_copy(ids_hbm.at[wid], idx_v)

    def fire_in(c, p):
        s0 = base + c * K
        pltpu.async_copy(pe_hbm.at[pl.ds(s0, K), :], pe_v.at[p], psem)
        pltpu.async_copy(
            table_hbm.at[idx_v.at[pl.ds(c * B * K, B * K)]],
            rows_v.at[p], gsem)

    def wait_in(c, p):
        s0 = base + c * K
        pltpu.make_async_copy(pe_hbm.at[pl.ds(s0, K), :], pe_v.at[p], psem).wait()
        pltpu.make_async_copy(
            table_hbm.at[idx_v.at[pl.ds(c * B * K, B * K)]],
            rows_v.at[p], gsem).wait()

    def fire_out(c, p):
        s0 = base + c * K
        for b in range(B):
            pltpu.async_copy(
                rows_v.at[p, pl.ds(b * K, K), :],
                out_hbm.at[b, pl.ds(s0, K), :], osem)

    def wait_out_one(p):
        for b in range(B):
            pltpu.make_async_copy(
                rows_v.at[p, pl.ds(b * K, K), :],
                out_hbm.at[b, pl.ds(0, K), :], osem).wait()

    fire_in(0, 0)

    @pl.loop(0, NCH, step=NBUF)
    def _group(c0):
        for ph in range(NBUF):
            c = c0 + ph
            nxt = (ph + 1) % NBUF

            @pl.when(c >= NBUF - 1)
            def _():
                wait_out_one(nxt)

            @pl.when(c + 1 < NCH)
            def _():
                fire_in(c + 1, nxt)

            wait_in(c, ph)

            @pl.loop(0, K)
            def _row(r):
                @pl.loop(0, HID // LANES, unroll=4)
                def _vec(j):
                    off = pl.ds(j * LANES, LANES)
                    pv = pe_v[ph, r, off]
                    for b in range(B):
                        rows_v[ph, b * K + r, off] = (
                            rows_v[ph, b * K + r, off] * SCALE + pv)

            fire_out(c, ph)

    for p in range(NBUF - 1):
        wait_out_one((NCH + p) % NBUF)


def kernel(token_ids, table, pe):
    ids = token_ids.astype(jnp.int32)
    ids_perm = (ids.reshape(B, NW, NCH, K)
                   .transpose(1, 2, 0, 3)
                   .reshape(NW, NCH * B * K))
    pe2 = pe.reshape(pe.shape[1], pe.shape[2])[:S]
    return _embed(ids_perm, table, pe2)


# speedup vs baseline: 1.7475x; 1.7475x over previous
import functools
import jax
import jax.numpy as jnp
from jax import lax
from jax.experimental import pallas as pl
from jax.experimental.pallas import tpu as pltpu
from jax.experimental.pallas import tpu_sc as plsc

B = 4
S = 8192
HID = 1024
NC = 2
NS = 16
NW = NC * NS
SW = S // NW
K = 8
NCH = SW // K
NBUF = 2
LANES = 16
SCALE = 32.0

_mesh = plsc.VectorSubcoreMesh(core_axis_name="c", subcore_axis_name="s")


@functools.partial(
    pl.kernel,
    out_type=jax.ShapeDtypeStruct((B, S, HID), jnp.float32),
    mesh=_mesh,
    scratch_types=[
        pltpu.VMEM((NCH * B * K,), jnp.int32),
        pltpu.VMEM((NBUF, K, HID), jnp.float32),
        pltpu.VMEM((NBUF, B * K, HID), jnp.float32),
        pltpu.SemaphoreType.DMA,
        pltpu.SemaphoreType.DMA,
        pltpu.SemaphoreType.DMA,
    ],
)
def _embed(ids_hbm, table_hbm, pe_hbm, out_hbm, idx_v, pe_v, rows_v,
           gsem, psem, osem):
    wid = lax.axis_index("s") * NC + lax.axis_index("c")
    base = wid * SW

    pltpu.sync_copy(ids_hbm.at[wid], idx_v)

    def fire_in(c, p):
        s0 = base + c * K
        pltpu.async_copy(pe_hbm.at[pl.ds(s0, K), :], pe_v.at[p], psem)
        pltpu.async_copy(
            table_hbm.at[idx_v.at[pl.ds(c * B * K, B * K)]],
            rows_v.at[p], gsem)

    def wait_in(c, p):
        s0 = base + c * K
        pltpu.make_async_copy(pe_hbm.at[pl.ds(s0, K), :], pe_v.at[p], psem).wait()
        pltpu.make_async_copy(
            table_hbm.at[idx_v.at[pl.ds(c * B * K, B * K)]],
            rows_v.at[p], gsem).wait()

    def fire_out(c, p):
        s0 = base + c * K
        for b in range(B):
            pltpu.async_copy(
                rows_v.at[p, pl.ds(b * K, K), :],
                out_hbm.at[b, pl.ds(s0, K), :], osem)

    def wait_out_one(p):
        for b in range(B):
            pltpu.make_async_copy(
                rows_v.at[p, pl.ds(b * K, K), :],
                out_hbm.at[b, pl.ds(0, K), :], osem).wait()

    fire_in(0, 0)

    @pl.loop(0, NCH, step=NBUF)
    def _group(c0):
        for ph in range(NBUF):
            c = c0 + ph
            nxt = (ph + 1) % NBUF

            @pl.when(c >= NBUF - 1)
            def _():
                wait_out_one(nxt)

            @pl.when(c + 1 < NCH)
            def _():
                fire_in(c + 1, nxt)

            wait_in(c, ph)

            @pl.loop(0, K)
            def _row(r):
                @plsc.parallel_loop(0, HID // LANES, unroll=8)
                def _vec(j):
                    off = pl.ds(j * LANES, LANES)
                    pv = pe_v[ph, r, off]
                    for b in range(B):
                        rows_v[ph, b * K + r, off] = (
                            rows_v[ph, b * K + r, off] * SCALE + pv)

            fire_out(c, ph)

    for p in range(NBUF - 1):
        wait_out_one((NCH + p) % NBUF)


def kernel(token_ids, table, pe):
    ids = token_ids.astype(jnp.int32)
    ids_perm = (ids.reshape(B, NW, NCH, K)
                   .transpose(1, 2, 0, 3)
                   .reshape(NW, NCH * B * K))
    pe2 = pe.reshape(pe.shape[1], pe.shape[2])[:S]
    return _embed(ids_perm, table, pe2)
